# baseline (speedup 1.0000x reference)
import jax
import jax.numpy as jnp
from jax import lax
from jax.experimental import pallas as pl
from jax.experimental.pallas import tpu as pltpu
from jax.experimental.pallas import tpu_sc as plsc

_NUM_CORES = 2
_NUM_SUBCORES = 16
_NW = _NUM_CORES * _NUM_SUBCORES

_B = 4096
_L = 200
_D = 64
_REPEATS = 2
_BPT = _B // _NW
_NBUF = 2


def _body(codet_hbm, table_hbm, out_hbm, idx_v, rows_bufs, trans_v, gsems, wsems):
    wid = lax.axis_index("s") * _NUM_CORES + lax.axis_index("c")
    bt = wid
    bt0 = wid * _BPT

    pltpu.sync_copy(codet_hbm.at[:, pl.ds(bt0, _BPT)], idx_v)

    def gather_copy(l, b):
        return pltpu.make_async_copy(
            table_hbm.at[idx_v.at[l]],
            rows_bufs[b],
            gsems[b],
        )

    def out_copies(l, b):
        for r in range(_REPEATS):
            yield pltpu.make_async_copy(
                trans_v.at[b],
                out_hbm.at[_REPEATS * l + r, :, bt],
                wsems[b],
            )

    for b in range(_NBUF):
        gather_copy(b, b).start()

    biota = [lax.iota(jnp.int32, 16) + 16 * bc for bc in range(_BPT // 16)]

    def outer(i, _):
        for b in range(_NBUF):
            l = i * _NBUF + b
            gather_copy(l, b).wait()

            @pl.when(l >= _NBUF)
            def _():
                for c in out_copies(l - _NBUF, b):
                    c.wait()

            def trans(d, _):
                dsplat = jnp.full((16,), d, jnp.int32)
                for bc in range(_BPT // 16):
                    v = plsc.load_gather(rows_bufs[b], [biota[bc], dsplat])
                    trans_v[b, d // 8, d % 8, pl.ds(bc * 16, 16)] = v
                return 0

            lax.fori_loop(0, _D, trans, 0, unroll=2)

            for c in out_copies(l, b):
                c.start()

            @pl.when(l + _NBUF < _L)
            def _():
                gather_copy(l + _NBUF, b).start()
        return 0

    lax.fori_loop(0, _L // _NBUF, outer, 0)

    for b in range(_NBUF):
        for c in out_copies(_L - _NBUF + b, b):
            c.wait()


@jax.jit
def _run(codet, table):
    k = pl.kernel(
        _body,
        out_type=jax.ShapeDtypeStruct(
            (_L * _REPEATS, _D // 8, _NW, 8, _BPT), jnp.float32),
        mesh=plsc.VectorSubcoreMesh(
            core_axis_name="c", subcore_axis_name="s",
            num_cores=_NUM_CORES, num_subcores=_NUM_SUBCORES,
        ),
        scratch_types=[
            pltpu.VMEM((_L, _BPT), jnp.int32),
            [pltpu.VMEM((_BPT, _D), jnp.float32)] * _NBUF,
            pltpu.VMEM((_NBUF, _D // 8, 8, _BPT), jnp.float32),
            [pltpu.SemaphoreType.DMA] * _NBUF,
            [pltpu.SemaphoreType.DMA] * _NBUF,
        ],
        compiler_params=pltpu.CompilerParams(
            use_tc_tiling_on_sc=False, needs_layout_passes=False),
    )
    return k(codet, table)


def kernel(code, table):
    codet = jnp.transpose(code).astype(jnp.int32)
    out5 = _run(codet, table)
    return jnp.transpose(out5, (2, 4, 0, 1, 3)).reshape(_B, _L * _REPEATS, _D)


# speedup vs baseline: 1.9910x; 1.9910x over previous
import jax
import jax.numpy as jnp
from jax import lax
from jax.experimental import pallas as pl
from jax.experimental.pallas import tpu as pltpu
from jax.experimental.pallas import tpu_sc as plsc

_NUM_CORES = 2
_NUM_SUBCORES = 16
_NW = _NUM_CORES * _NUM_SUBCORES

_B = 4096
_L = 200
_D = 64
_REPEATS = 2
_BPT = _B // _NW
_NBUF = 2


def _body(codet_hbm, table_hbm, out_hbm, idx_v, rows_bufs, trans_bufs,
          gsems, wsems):
    wid = lax.axis_index("s") * _NUM_CORES + lax.axis_index("c")
    bt = wid
    bt0 = wid * _BPT

    pltpu.sync_copy(codet_hbm.at[:, pl.ds(bt0, _BPT)], idx_v)

    def gather_copy(l, b):
        return pltpu.make_async_copy(
            table_hbm.at[idx_v.at[l]],
            rows_bufs[b],
            gsems[b],
        )

    def out_copies(l, b):
        for r in range(_REPEATS):
            yield pltpu.make_async_copy(
                trans_bufs[b],
                out_hbm.at[_REPEATS * l + r, :, bt],
                wsems[b],
            )

    for b in range(_NBUF):
        gather_copy(b, b).start()

    iota16 = lax.iota(jnp.int32, 16)
    perm = [jnp.bitwise_and(iota16 + k, 15) for k in range(16)]

    def outer(i, _):
        for b in range(_NBUF):
            l = i * _NBUF + b
            gather_copy(l, b).wait()

            @pl.when(l >= _NBUF)
            def _():
                for c in out_copies(l - _NBUF, b):
                    c.wait()

            def trans(bc, _):
                b0 = bc * 16
                bvec = iota16 + b0
                for dc in range(_D // 16):
                    for k in range(16):
                        dvec = perm[k] + (dc * 16)
                        v = plsc.load_gather(rows_bufs[b], [bvec, dvec])
                        plsc.store_scatter(
                            trans_bufs[b],
                            [lax.shift_right_logical(dvec, 3),
                             jnp.bitwise_and(dvec, 7), bvec],
                            v)
                return 0

            lax.fori_loop(0, _BPT // 16, trans, 0)

            for c in out_copies(l, b):
                c.start()

            @pl.when(l + _NBUF < _L)
            def _():
                gather_copy(l + _NBUF, b).start()
        return 0

    lax.fori_loop(0, _L // _NBUF, outer, 0)

    for b in range(_NBUF):
        for c in out_copies(_L - _NBUF + b, b):
            c.wait()


@jax.jit
def _run(codet, table):
    k = pl.kernel(
        _body,
        out_type=jax.ShapeDtypeStruct(
            (_L * _REPEATS, _D // 8, _NW, 8, _BPT), jnp.float32),
        mesh=plsc.VectorSubcoreMesh(
            core_axis_name="c", subcore_axis_name="s",
            num_cores=_NUM_CORES, num_subcores=_NUM_SUBCORES,
        ),
        scratch_types=[
            pltpu.VMEM((_L, _BPT), jnp.int32),
            [pltpu.VMEM((_BPT, _D), jnp.float32)] * _NBUF,
            [pltpu.VMEM((_D // 8, 8, _BPT), jnp.float32)] * _NBUF,
            [pltpu.SemaphoreType.DMA] * _NBUF,
            [pltpu.SemaphoreType.DMA] * _NBUF,
        ],
        compiler_params=pltpu.CompilerParams(
            use_tc_tiling_on_sc=False, needs_layout_passes=False),
    )
    return k(codet, table)


def kernel(code, table):
    codet = jnp.transpose(code).astype(jnp.int32)
    out5 = _run(codet, table)
    return jnp.transpose(out5, (2, 4, 0, 1, 3)).reshape(_B, _L * _REPEATS, _D)
